# baseline (speedup 1.0000x reference)
import jax
import jax.numpy as jnp
from jax import lax
from jax.experimental import pallas as pl
from jax.experimental.pallas import tpu as pltpu
from jax.experimental.pallas import tpu_sc as plsc

N = 2048
D = 768
H = 3072
E = 8
BR = 256
MAXB = 24
MAXR = MAXB * BR
BH = 1536
NH = H // BH
NC = 2
NS = 16
NW = NC * NS
ACH = 2 * N // NW
TCH = N // NW
_INV_SQRT2 = 0.7071067811865476


def _routing_body(x_ref, gw_ref, gb_ref, p_ref, ws_ref, be_ref, x16_ref):
    x = x_ref[...]
    x16_ref[...] = x.astype(jnp.bfloat16)
    g = jnp.dot(x, gw_ref[...], preferred_element_type=jnp.float32) + gb_ref[...]
    iota8 = lax.broadcasted_iota(jnp.int32, (N, E), 1)
    m1 = jnp.max(g, axis=1, keepdims=True)
    i1 = jnp.min(jnp.where(g == m1, iota8, E), axis=1, keepdims=True)
    g2 = jnp.where(iota8 == i1, -jnp.inf, g)
    m2 = jnp.max(g2, axis=1, keepdims=True)
    i2 = jnp.min(jnp.where(g2 == m2, iota8, E), axis=1, keepdims=True)
    t = jnp.exp(m2 - m1)
    ws1 = 1.0 / (1.0 + t)
    ws2 = t / (1.0 + t)
    a = jnp.concatenate([i1, i2], axis=0)
    oh = (a == lax.broadcasted_iota(jnp.int32, (2 * N, E), 1)).astype(jnp.int32)
    C = oh
    k = 1
    while k < 2 * N:
        C = C + jnp.concatenate([jnp.zeros((k, E), jnp.int32), C[: 2 * N - k]], axis=0)
        k *= 2
    counts = C[2 * N - 1 : 2 * N, :]
    nblk = (counts + BR - 1) // BR
    tri = (lax.broadcasted_iota(jnp.int32, (E, E), 0)
           <= lax.broadcasted_iota(jnp.int32, (E, E), 1)).astype(jnp.float32)
    ends = jnp.dot(nblk.astype(jnp.float32), tri,
                   preferred_element_type=jnp.float32).astype(jnp.int32)
    starts = ends - nblk
    rowstart = starts * BR
    rank = jnp.sum(C * oh, axis=1, keepdims=True) - 1
    pstart = jnp.sum(oh * rowstart, axis=1, keepdims=True)
    p_ref[...] = pstart + rank
    ws_ref[...] = jnp.concatenate([ws1, ws2], axis=0)
    j32 = lax.broadcasted_iota(jnp.int32, (32, E), 0)
    be_ref[...] = jnp.sum((j32 >= ends).astype(jnp.int32), axis=1, keepdims=True)


def _routing(x2, gate_w, gate_b2):
    return pl.pallas_call(
        _routing_body,
        out_shape=[
            jax.ShapeDtypeStruct((2 * N, 1), jnp.int32),
            jax.ShapeDtypeStruct((2 * N, 1), jnp.float32),
            jax.ShapeDtypeStruct((32, 1), jnp.int32),
            jax.ShapeDtypeStruct((N, D), jnp.bfloat16),
        ],
    )(x2, gate_w, gate_b2)


def _dispatch_body(x_hbm, p_hbm, xs_hbm, idx_v, rows_v, sem_s, sem_g):
    wid = lax.axis_index("s") * NC + lax.axis_index("c")
    base = wid * ACH
    tok = lax.rem(base, N)
    pltpu.sync_copy(p_hbm.at[pl.ds(base, ACH)], idx_v)
    pltpu.async_copy(x_hbm.at[pl.ds(tok, ACH)], rows_v, sem_g).wait()
    pltpu.async_copy(rows_v, xs_hbm.at[idx_v], sem_s).wait()


def _dispatch(x2, p):
    call = pl.kernel(
        _dispatch_body,
        out_type=jax.ShapeDtypeStruct((MAXR, D // 2), jnp.int32),
        mesh=plsc.VectorSubcoreMesh(core_axis_name="c", subcore_axis_name="s",
                                    num_cores=NC, num_subcores=NS),
        scratch_types=[
            pltpu.VMEM((ACH,), jnp.int32),
            pltpu.VMEM((ACH, D // 2), jnp.int32),
            pltpu.SemaphoreType.DMA,
            pltpu.SemaphoreType.DMA,
        ],
    )
    return call(x2, p)


def _ffn_body(be_ref, xs_ref, w1_ref, b1_ref, w2_ref, b2_ref, ys_ref):
    r = pl.program_id(0)

    @pl.when(be_ref[r] < E)
    def _():
        xb = xs_ref[...].astype(jnp.float32)
        hpre = jnp.dot(xb, w1_ref[0], preferred_element_type=jnp.float32) + b1_ref[0]
        hact = 0.5 * hpre * (1.0 + lax.erf(hpre * _INV_SQRT2))
        part = jnp.dot(hact, w2_ref[0], preferred_element_type=jnp.float32)
        ys_ref[...] = (part + b2_ref[0]).astype(jnp.bfloat16)


def _ffn(be, xs, w1, b1, w2, b2):
    def wexp(r, be_ref):
        return (jnp.minimum(be_ref[r], E - 1), 0, 0)

    grid_spec = pltpu.PrefetchScalarGridSpec(
        num_scalar_prefetch=1,
        grid=(MAXB,),
        in_specs=[
            pl.BlockSpec((BR, D), lambda r, be_ref: (r, 0)),
            pl.BlockSpec((1, D, H), wexp),
            pl.BlockSpec((1, 1, H), wexp),
            pl.BlockSpec((1, H, D), wexp),
            pl.BlockSpec((1, 1, D), wexp),
        ],
        out_specs=pl.BlockSpec((BR, D), lambda r, be_ref: (r, 0)),
    )
    return pl.pallas_call(
        _ffn_body,
        grid_spec=grid_spec,
        out_shape=jax.ShapeDtypeStruct((MAXR, D), jnp.bfloat16),
        compiler_params=pltpu.CompilerParams(
            dimension_semantics=("arbitrary",)),
    )(be, xs, w1, b1, w2, b2)


def _gather_body(ys_hbm, p_hbm, a_hbm, b_hbm, i1_v, i2_v, a_v, b_v, sa, sb):
    wid = lax.axis_index("s") * NC + lax.axis_index("c")
    base = wid * TCH
    pltpu.sync_copy(p_hbm.at[pl.ds(base, TCH)], i1_v)
    pltpu.sync_copy(p_hbm.at[pl.ds(N + base, TCH)], i2_v)
    da = pltpu.async_copy(ys_hbm.at[i1_v], a_v, sa)
    db = pltpu.async_copy(ys_hbm.at[i2_v], b_v, sb)
    da.wait()
    db.wait()
    pltpu.sync_copy(a_v, a_hbm.at[pl.ds(base, TCH)])
    pltpu.sync_copy(b_v, b_hbm.at[pl.ds(base, TCH)])


def _gather2(ys, p):
    call = pl.kernel(
        _gather_body,
        out_type=[
            jax.ShapeDtypeStruct((N, D // 2), jnp.int32),
            jax.ShapeDtypeStruct((N, D // 2), jnp.int32),
        ],
        mesh=plsc.VectorSubcoreMesh(core_axis_name="c", subcore_axis_name="s",
                                    num_cores=NC, num_subcores=NS),
        scratch_types=[
            pltpu.VMEM((TCH,), jnp.int32),
            pltpu.VMEM((TCH,), jnp.int32),
            pltpu.VMEM((TCH, D // 2), jnp.int32),
            pltpu.VMEM((TCH, D // 2), jnp.int32),
            pltpu.SemaphoreType.DMA,
            pltpu.SemaphoreType.DMA,
        ],
    )
    return call(ys, p)


def _combine_body(a_ref, b_ref, ws_ref, out_ref):
    out_ref[...] = (a_ref[...].astype(jnp.float32) * ws_ref[:, 0:1]
                    + b_ref[...].astype(jnp.float32) * ws_ref[:, 1:2])


def _combine(a, b, ws):
    return pl.pallas_call(
        _combine_body,
        grid=(N // BR,),
        in_specs=[
            pl.BlockSpec((BR, D), lambda r: (r, 0)),
            pl.BlockSpec((BR, D), lambda r: (r, 0)),
            pl.BlockSpec((BR, 2), lambda r: (r, 0)),
        ],
        out_specs=pl.BlockSpec((BR, D), lambda r: (r, 0)),
        out_shape=jax.ShapeDtypeStruct((N, D), jnp.float32),
    )(a, b, ws)


def _to_i32(v16):
    return lax.bitcast_convert_type(
        v16.reshape(v16.shape[0], v16.shape[1] // 2, 2), jnp.int32)


def _to_bf16(vi32):
    v = lax.bitcast_convert_type(vi32, jnp.bfloat16)
    return v.reshape(v.shape[0], v.shape[1] * 2)


def kernel(x, gate_w, gate_b, w1, b1, w2, b2):
    x2 = x.reshape(N, D)
    p, ws, be, x16 = _routing(x2, gate_w, gate_b.reshape(1, E))
    p = p.reshape(2 * N)
    ws2c = ws.reshape(2, N).T
    xs = _dispatch(_to_i32(x16), p)
    ys = _ffn(be.reshape(32), _to_bf16(xs), w1, b1.reshape(E, 1, H), w2,
              b2.reshape(E, 1, D))
    a, b = _gather2(_to_i32(ys), p)
    out = _combine(_to_bf16(a), _to_bf16(b), ws2c)
    return out.reshape(1, N, D)


# speedup vs baseline: 2.9500x; 2.9500x over previous
import jax
import jax.numpy as jnp
from jax import lax
from jax.experimental import pallas as pl
from jax.experimental.pallas import tpu as pltpu
from jax.experimental.pallas import tpu_sc as plsc

N = 2048
D = 768
H = 3072
E = 8
BR = 256
MAXB = 24
MAXR = MAXB * BR
BH = 1536
NH = H // BH
NC = 2
NS = 16
NW = NC * NS
ACH = 2 * N // NW
TCH = N // NW
_INV_SQRT2 = 0.7071067811865476


def _routing_body(x_ref, gw_ref, gb_ref, p_ref, ws_ref, be_ref):
    x = x_ref[...]
    g = jnp.dot(x, gw_ref[...], preferred_element_type=jnp.float32) + gb_ref[...]
    iota8 = lax.broadcasted_iota(jnp.int32, (N, E), 1)
    m1 = jnp.max(g, axis=1, keepdims=True)
    i1 = jnp.min(jnp.where(g == m1, iota8, E), axis=1, keepdims=True)
    g2 = jnp.where(iota8 == i1, -jnp.inf, g)
    m2 = jnp.max(g2, axis=1, keepdims=True)
    i2 = jnp.min(jnp.where(g2 == m2, iota8, E), axis=1, keepdims=True)
    t = jnp.exp(m2 - m1)
    ws1 = 1.0 / (1.0 + t)
    ws2 = t / (1.0 + t)
    a = jnp.concatenate([i1, i2], axis=0)
    oh = (a == lax.broadcasted_iota(jnp.int32, (2 * N, E), 1)).astype(jnp.int32)
    C = oh
    k = 1
    while k < 2 * N:
        C = C + jnp.concatenate([jnp.zeros((k, E), jnp.int32), C[: 2 * N - k]], axis=0)
        k *= 2
    counts = C[2 * N - 1 : 2 * N, :]
    nblk = (counts + BR - 1) // BR
    tri = (lax.broadcasted_iota(jnp.int32, (E, E), 0)
           <= lax.broadcasted_iota(jnp.int32, (E, E), 1)).astype(jnp.float32)
    ends = jnp.dot(nblk.astype(jnp.float32), tri,
                   preferred_element_type=jnp.float32).astype(jnp.int32)
    starts = ends - nblk
    rowstart = starts * BR
    rank = jnp.sum(C * oh, axis=1, keepdims=True) - 1
    pstart = jnp.sum(oh * rowstart, axis=1, keepdims=True)
    p_ref[...] = pstart + rank
    ws_ref[...] = jnp.concatenate([ws1, ws2], axis=0)
    j32 = lax.broadcasted_iota(jnp.int32, (32, E), 0)
    be_ref[...] = jnp.sum((j32 >= ends).astype(jnp.int32), axis=1, keepdims=True)


def _routing(x2, gate_w, gate_b2):
    return pl.pallas_call(
        _routing_body,
        out_shape=[
            jax.ShapeDtypeStruct((2 * N, 1), jnp.int32),
            jax.ShapeDtypeStruct((2 * N, 1), jnp.float32),
            jax.ShapeDtypeStruct((32, 1), jnp.int32),
        ],
    )(x2, gate_w, gate_b2)


def _dispatch_body(x_hbm, p_hbm, xs_hbm, idx_v, rows_v, sem_s, sem_g):
    wid = lax.axis_index("s") * NC + lax.axis_index("c")
    base = wid * ACH
    tok = lax.rem(base, N)
    pltpu.sync_copy(p_hbm.at[pl.ds(base, ACH)], idx_v)
    pltpu.async_copy(x_hbm.at[pl.ds(tok, ACH)], rows_v, sem_g).wait()
    pltpu.async_copy(rows_v, xs_hbm.at[idx_v], sem_s).wait()


def _dispatch(x2, p):
    call = pl.kernel(
        _dispatch_body,
        out_type=jax.ShapeDtypeStruct((MAXR, D), jnp.float32),
        mesh=plsc.VectorSubcoreMesh(core_axis_name="c", subcore_axis_name="s",
                                    num_cores=NC, num_subcores=NS),
        scratch_types=[
            pltpu.VMEM((ACH,), jnp.int32),
            pltpu.VMEM((ACH, D), jnp.float32),
            pltpu.SemaphoreType.DMA,
            pltpu.SemaphoreType.DMA,
        ],
    )
    return call(x2, p)


def _ffn_body(be_ref, xs_ref, w1_ref, b1_ref, w2_ref, b2_ref, ys_ref):
    r = pl.program_id(0)

    @pl.when(be_ref[r] < E)
    def _():
        xb = xs_ref[...]
        hpre = jnp.dot(xb, w1_ref[0], preferred_element_type=jnp.float32) + b1_ref[0]
        hact = 0.5 * hpre * (1.0 + lax.erf(hpre * _INV_SQRT2))
        part = jnp.dot(hact, w2_ref[0], preferred_element_type=jnp.float32)
        ys_ref[...] = part + b2_ref[0]


def _ffn(be, xs, w1, b1, w2, b2):
    def wexp(r, be_ref):
        return (jnp.minimum(be_ref[r], E - 1), 0, 0)

    grid_spec = pltpu.PrefetchScalarGridSpec(
        num_scalar_prefetch=1,
        grid=(MAXB,),
        in_specs=[
            pl.BlockSpec((BR, D), lambda r, be_ref: (r, 0)),
            pl.BlockSpec((1, D, H), wexp),
            pl.BlockSpec((1, 1, H), wexp),
            pl.BlockSpec((1, H, D), wexp),
            pl.BlockSpec((1, 1, D), wexp),
        ],
        out_specs=pl.BlockSpec((BR, D), lambda r, be_ref: (r, 0)),
    )
    return pl.pallas_call(
        _ffn_body,
        grid_spec=grid_spec,
        out_shape=jax.ShapeDtypeStruct((MAXR, D), jnp.float32),
        compiler_params=pltpu.CompilerParams(
            dimension_semantics=("arbitrary",)),
    )(be, xs, w1, b1, w2, b2)


def _gather_body(ys_hbm, p_hbm, a_hbm, b_hbm, i1_v, i2_v, a_v, b_v, sa, sb):
    wid = lax.axis_index("s") * NC + lax.axis_index("c")
    base = wid * TCH
    pltpu.sync_copy(p_hbm.at[pl.ds(base, TCH)], i1_v)
    pltpu.sync_copy(p_hbm.at[pl.ds(N + base, TCH)], i2_v)
    da = pltpu.async_copy(ys_hbm.at[i1_v], a_v, sa)
    db = pltpu.async_copy(ys_hbm.at[i2_v], b_v, sb)
    da.wait()
    db.wait()
    pltpu.sync_copy(a_v, a_hbm.at[pl.ds(base, TCH)])
    pltpu.sync_copy(b_v, b_hbm.at[pl.ds(base, TCH)])


def _gather2(ys, p):
    call = pl.kernel(
        _gather_body,
        out_type=[
            jax.ShapeDtypeStruct((N, D), jnp.float32),
            jax.ShapeDtypeStruct((N, D), jnp.float32),
        ],
        mesh=plsc.VectorSubcoreMesh(core_axis_name="c", subcore_axis_name="s",
                                    num_cores=NC, num_subcores=NS),
        scratch_types=[
            pltpu.VMEM((TCH,), jnp.int32),
            pltpu.VMEM((TCH,), jnp.int32),
            pltpu.VMEM((TCH, D), jnp.float32),
            pltpu.VMEM((TCH, D), jnp.float32),
            pltpu.SemaphoreType.DMA,
            pltpu.SemaphoreType.DMA,
        ],
    )
    return call(ys, p)


def _combine_body(a_ref, b_ref, w1_ref, w2_ref, out_ref):
    out_ref[...] = a_ref[...] * w1_ref[...] + b_ref[...] * w2_ref[...]


def _combine(a, b, ws1, ws2):
    return pl.pallas_call(
        _combine_body,
        grid=(N // BR,),
        in_specs=[
            pl.BlockSpec((BR, D), lambda r: (r, 0)),
            pl.BlockSpec((BR, D), lambda r: (r, 0)),
            pl.BlockSpec((BR, 1), lambda r: (r, 0)),
            pl.BlockSpec((BR, 1), lambda r: (r, 0)),
        ],
        out_specs=pl.BlockSpec((BR, D), lambda r: (r, 0)),
        out_shape=jax.ShapeDtypeStruct((N, D), jnp.float32),
    )(a, b, ws1, ws2)


def kernel(x, gate_w, gate_b, w1, b1, w2, b2):
    x2 = x.reshape(N, D)
    p, ws, be = _routing(x2, gate_w, gate_b.reshape(1, E))
    p = p.reshape(2 * N)
    xs = _dispatch(x2, p)
    ys = _ffn(be.reshape(32), xs, w1, b1.reshape(E, 1, H), w2,
              b2.reshape(E, 1, D))
    a, b = _gather2(ys, p)
    out = _combine(a, b, ws[:N], ws[N:])
    return out.reshape(1, N, D)


# speedup vs baseline: 3.0846x; 1.0456x over previous
import jax
import jax.numpy as jnp
from jax import lax
from jax.experimental import pallas as pl
from jax.experimental.pallas import tpu as pltpu
from jax.experimental.pallas import tpu_sc as plsc

N = 2048
D = 768
H = 3072
E = 8
BR = 256
MAXB = 24
MAXR = MAXB * BR
BH = 1536
NH = H // BH
NC = 2
NS = 16
NW = NC * NS
ACH = 2 * N // NW
TCH = N // NW
HD = D // 2
_INV_SQRT2 = 0.7071067811865476


def _pack(v):
    lo = lax.bitcast_convert_type(v[:, :HD].astype(jnp.bfloat16), jnp.uint16)
    hi = lax.bitcast_convert_type(v[:, HD:].astype(jnp.bfloat16), jnp.uint16)
    packed = lo.astype(jnp.uint32) | (hi.astype(jnp.uint32) << 16)
    return lax.bitcast_convert_type(packed, jnp.int32)


def _unpack(p):
    u = lax.bitcast_convert_type(p, jnp.uint32)
    lo = lax.bitcast_convert_type((u & jnp.uint32(0xFFFF)).astype(jnp.uint16),
                                  jnp.bfloat16)
    hi = lax.bitcast_convert_type((u >> 16).astype(jnp.uint16), jnp.bfloat16)
    return lo.astype(jnp.float32), hi.astype(jnp.float32)


def _routing_body(x_ref, gw_ref, gb_ref, p_ref, ws_ref, be_ref, xp_ref):
    x = x_ref[...]
    xp_ref[...] = _pack(x)
    g = jnp.dot(x, gw_ref[...], preferred_element_type=jnp.float32) + gb_ref[...]
    iota8 = lax.broadcasted_iota(jnp.int32, (N, E), 1)
    m1 = jnp.max(g, axis=1, keepdims=True)
    i1 = jnp.min(jnp.where(g == m1, iota8, E), axis=1, keepdims=True)
    g2 = jnp.where(iota8 == i1, -jnp.inf, g)
    m2 = jnp.max(g2, axis=1, keepdims=True)
    i2 = jnp.min(jnp.where(g2 == m2, iota8, E), axis=1, keepdims=True)
    t = jnp.exp(m2 - m1)
    ws1 = 1.0 / (1.0 + t)
    ws2 = t / (1.0 + t)
    a = jnp.concatenate([i1, i2], axis=0)
    oh = (a == lax.broadcasted_iota(jnp.int32, (2 * N, E), 1)).astype(jnp.int32)
    C = oh
    k = 1
    while k < 2 * N:
        C = C + jnp.concatenate([jnp.zeros((k, E), jnp.int32), C[: 2 * N - k]], axis=0)
        k *= 2
    counts = C[2 * N - 1 : 2 * N, :]
    nblk = (counts + BR - 1) // BR
    tri = (lax.broadcasted_iota(jnp.int32, (E, E), 0)
           <= lax.broadcasted_iota(jnp.int32, (E, E), 1)).astype(jnp.float32)
    ends = jnp.dot(nblk.astype(jnp.float32), tri,
                   preferred_element_type=jnp.float32).astype(jnp.int32)
    starts = ends - nblk
    rowstart = starts * BR
    rank = jnp.sum(C * oh, axis=1, keepdims=True) - 1
    pstart = jnp.sum(oh * rowstart, axis=1, keepdims=True)
    p_ref[...] = pstart + rank
    ws_ref[...] = jnp.concatenate([ws1, ws2], axis=0)
    j32 = lax.broadcasted_iota(jnp.int32, (32, E), 0)
    be_ref[...] = jnp.sum((j32 >= ends).astype(jnp.int32), axis=1, keepdims=True)


def _routing(x2, gate_w, gate_b2):
    return pl.pallas_call(
        _routing_body,
        out_shape=[
            jax.ShapeDtypeStruct((2 * N, 1), jnp.int32),
            jax.ShapeDtypeStruct((2 * N, 1), jnp.float32),
            jax.ShapeDtypeStruct((32, 1), jnp.int32),
            jax.ShapeDtypeStruct((N, HD), jnp.int32),
        ],
    )(x2, gate_w, gate_b2)


def _dispatch_body(x_hbm, p_hbm, xs_hbm, idx_v, rows_v, sem_s, sem_g):
    wid = lax.axis_index("s") * NC + lax.axis_index("c")
    base = wid * ACH
    tok = lax.rem(base, N)
    pltpu.sync_copy(p_hbm.at[pl.ds(base, ACH)], idx_v)
    pltpu.async_copy(x_hbm.at[pl.ds(tok, ACH)], rows_v, sem_g).wait()
    pltpu.async_copy(rows_v, xs_hbm.at[idx_v], sem_s).wait()


def _dispatch(x2, p):
    call = pl.kernel(
        _dispatch_body,
        out_type=jax.ShapeDtypeStruct((MAXR, HD), jnp.int32),
        mesh=plsc.VectorSubcoreMesh(core_axis_name="c", subcore_axis_name="s",
                                    num_cores=NC, num_subcores=NS),
        scratch_types=[
            pltpu.VMEM((ACH,), jnp.int32),
            pltpu.VMEM((ACH, HD), jnp.int32),
            pltpu.SemaphoreType.DMA,
            pltpu.SemaphoreType.DMA,
        ],
    )
    return call(x2, p)


def _ffn_body(be_ref, xs_ref, w1_ref, b1_ref, w2_ref, b2_ref, ys_ref):
    r = pl.program_id(0)

    @pl.when(be_ref[r] < E)
    def _():
        xa, xb = _unpack(xs_ref[...])
        hpre = (jnp.dot(xa, w1_ref[0, :HD, :], preferred_element_type=jnp.float32)
                + jnp.dot(xb, w1_ref[0, HD:, :], preferred_element_type=jnp.float32)
                + b1_ref[0])
        hact = 0.5 * hpre * (1.0 + lax.erf(hpre * _INV_SQRT2))
        part = jnp.dot(hact, w2_ref[0], preferred_element_type=jnp.float32)
        ys_ref[...] = _pack(part + b2_ref[0])


def _ffn(be, xs, w1, b1, w2, b2):
    def wexp(r, be_ref):
        return (jnp.minimum(be_ref[r], E - 1), 0, 0)

    grid_spec = pltpu.PrefetchScalarGridSpec(
        num_scalar_prefetch=1,
        grid=(MAXB,),
        in_specs=[
            pl.BlockSpec((BR, HD), lambda r, be_ref: (r, 0)),
            pl.BlockSpec((1, D, H), wexp),
            pl.BlockSpec((1, 1, H), wexp),
            pl.BlockSpec((1, H, D), wexp),
            pl.BlockSpec((1, 1, D), wexp),
        ],
        out_specs=pl.BlockSpec((BR, HD), lambda r, be_ref: (r, 0)),
    )
    return pl.pallas_call(
        _ffn_body,
        grid_spec=grid_spec,
        out_shape=jax.ShapeDtypeStruct((MAXR, HD), jnp.int32),
        compiler_params=pltpu.CompilerParams(
            dimension_semantics=("arbitrary",)),
    )(be, xs, w1, b1, w2, b2)


def _gather_body(ys_hbm, p_hbm, a_hbm, b_hbm, i1_v, i2_v, a_v, b_v, sa, sb):
    wid = lax.axis_index("s") * NC + lax.axis_index("c")
    base = wid * TCH
    pltpu.sync_copy(p_hbm.at[pl.ds(base, TCH)], i1_v)
    pltpu.sync_copy(p_hbm.at[pl.ds(N + base, TCH)], i2_v)
    da = pltpu.async_copy(ys_hbm.at[i1_v], a_v, sa)
    db = pltpu.async_copy(ys_hbm.at[i2_v], b_v, sb)
    da.wait()
    db.wait()
    pltpu.sync_copy(a_v, a_hbm.at[pl.ds(base, TCH)])
    pltpu.sync_copy(b_v, b_hbm.at[pl.ds(base, TCH)])


def _gather2(ys, p):
    call = pl.kernel(
        _gather_body,
        out_type=[
            jax.ShapeDtypeStruct((N, HD), jnp.int32),
            jax.ShapeDtypeStruct((N, HD), jnp.int32),
        ],
        mesh=plsc.VectorSubcoreMesh(core_axis_name="c", subcore_axis_name="s",
                                    num_cores=NC, num_subcores=NS),
        scratch_types=[
            pltpu.VMEM((TCH,), jnp.int32),
            pltpu.VMEM((TCH,), jnp.int32),
            pltpu.VMEM((TCH, HD), jnp.int32),
            pltpu.VMEM((TCH, HD), jnp.int32),
            pltpu.SemaphoreType.DMA,
            pltpu.SemaphoreType.DMA,
        ],
    )
    return call(ys, p)


def _combine_body(a_ref, b_ref, w1_ref, w2_ref, out_ref):
    aa, ab = _unpack(a_ref[...])
    ba, bb = _unpack(b_ref[...])
    w1c = w1_ref[...]
    w2c = w2_ref[...]
    out_ref[:, :HD] = aa * w1c + ba * w2c
    out_ref[:, HD:] = ab * w1c + bb * w2c


def _combine(a, b, ws1, ws2):
    return pl.pallas_call(
        _combine_body,
        grid=(N // BR,),
        in_specs=[
            pl.BlockSpec((BR, HD), lambda r: (r, 0)),
            pl.BlockSpec((BR, HD), lambda r: (r, 0)),
            pl.BlockSpec((BR, 1), lambda r: (r, 0)),
            pl.BlockSpec((BR, 1), lambda r: (r, 0)),
        ],
        out_specs=pl.BlockSpec((BR, D), lambda r: (r, 0)),
        out_shape=jax.ShapeDtypeStruct((N, D), jnp.float32),
    )(a, b, ws1, ws2)


def kernel(x, gate_w, gate_b, w1, b1, w2, b2):
    x2 = x.reshape(N, D)
    p, ws, be, xp = _routing(x2, gate_w, gate_b.reshape(1, E))
    p = p.reshape(2 * N)
    xs = _dispatch(xp, p)
    ys = _ffn(be.reshape(32), xs, w1, b1.reshape(E, 1, H), w2,
              b2.reshape(E, 1, D))
    a, b = _gather2(ys, p)
    out = _combine(a, b, ws[:N], ws[N:])
    return out.reshape(1, N, D)
